# Initial kernel scaffold; baseline (speedup 1.0000x reference)
import jax
import jax.numpy as jnp
from jax.experimental import pallas as pl


def kernel(x, edge_index, W1, b1, W2, b2, Wc0, bc0, Wc1, bc1, Wc2, bc2):
    raise NotImplementedError("write your pallas kernel here")


# speedup vs baseline: 19.8508x; 19.8508x over previous
import functools

import jax
import jax.numpy as jnp
from jax import lax
from jax.experimental import pallas as pl
from jax.experimental.pallas import tpu as pltpu
from jax.experimental.pallas import tpu_sc as plsc

NC = 2
NS = 16
LANE = 128
CH = 8

_HIGH = jax.lax.Precision.HIGHEST


def _mesh():
    return plsc.VectorSubcoreMesh(
        core_axis_name="c", subcore_axis_name="s", num_cores=NC, num_subcores=NS
    )


_SC_PARAMS = pltpu.CompilerParams(use_tc_tiling_on_sc=False)


def _sc_degree(dstr, ones_hbm, zeros_hbm, n_out, zb):
    rt = dstr.shape[0]
    rows_w = rt // (NC * NS)
    nch = rows_w // CH
    zr = n_out // NS
    out_r = n_out // NS
    zfull, zrem = zr // zb, zr % zb

    def body(dstr_h, ones_h, zeros_h, out_h, dst_v, ones_v, zbuf, deg_sh):
        c = lax.axis_index("c")
        s = lax.axis_index("s")
        pltpu.sync_copy(zeros_h, zbuf)
        pltpu.sync_copy(ones_h, ones_v)
        for z in range(zfull):
            pltpu.sync_copy(zbuf, deg_sh.at[pl.ds(s * zr + z * zb, zb)])
        if zrem:
            pltpu.sync_copy(zbuf.at[pl.ds(0, zrem)],
                            deg_sh.at[pl.ds(s * zr + zfull * zb, zrem)])
        plsc.subcore_barrier()
        base = (c * NS + s) * rows_w

        def chunk(i, _):
            pltpu.sync_copy(dstr_h.at[pl.ds(base + i * CH, CH)], dst_v)
            for j in range(CH):
                pltpu.sync_copy(ones_v, deg_sh.at[dst_v.at[j]], add=True)
            return ()

        lax.fori_loop(0, nch, chunk, ())
        plsc.subcore_barrier()
        pltpu.sync_copy(
            deg_sh.at[pl.ds(s * out_r, out_r)], out_h.at[c, pl.ds(s * out_r, out_r)]
        )

    call = pl.kernel(
        body,
        out_type=jax.ShapeDtypeStruct((NC, n_out, 16), jnp.float32),
        mesh=_mesh(),
        scratch_types=[
            pltpu.VMEM((CH, LANE), jnp.int32),
            pltpu.VMEM((LANE, 16), jnp.float32),
            pltpu.VMEM((zb, 16), jnp.float32),
            pltpu.VMEM_SHARED((n_out, 16), jnp.float32),
        ],
        compiler_params=_SC_PARAMS,
    )
    return call(dstr, ones_hbm, zeros_hbm)


def _sc_layer(g2, srcx, dstr, zeros_hbm, n_out):
    rt = dstr.shape[0]
    rows_w = rt // NS
    nch = rows_w // CH
    zr = n_out // NS
    out_r = n_out // NS
    zb = CH * LANE
    zfull, zrem = zr // zb, zr % zb

    def body(g2_h, srcx_h, dstr_h, zeros_h, out_h,
             src_v, dst_v, rows_v, acc_sh, sem):
        c = lax.axis_index("c")
        s = lax.axis_index("s")
        pltpu.sync_copy(zeros_h, rows_v)
        for z in range(zfull):
            pltpu.sync_copy(rows_v, acc_sh.at[pl.ds(s * zr + z * zb, zb)])
        if zrem:
            pltpu.sync_copy(rows_v.at[pl.ds(0, zrem)],
                            acc_sh.at[pl.ds(s * zr + zfull * zb, zrem)])
        plsc.subcore_barrier()
        base = s * rows_w

        def chunk(i, _):
            rb = base + i * CH
            pltpu.sync_copy(srcx_h.at[c, pl.ds(rb, CH)], src_v)
            pltpu.sync_copy(dstr_h.at[pl.ds(rb, CH)], dst_v)
            cps = [
                pltpu.async_copy(
                    g2_h.at[src_v.at[j]], rows_v.at[pl.ds(j * LANE, LANE)], sem
                )
                for j in range(CH)
            ]
            for cp in cps:
                cp.wait()
            for j in range(CH):
                pltpu.sync_copy(
                    rows_v.at[pl.ds(j * LANE, LANE)],
                    acc_sh.at[dst_v.at[j]],
                    add=True,
                )
            return ()

        lax.fori_loop(0, nch, chunk, ())
        plsc.subcore_barrier()
        pltpu.sync_copy(
            acc_sh.at[pl.ds(s * out_r, out_r)], out_h.at[c, pl.ds(s * out_r, out_r)]
        )

    call = pl.kernel(
        body,
        out_type=jax.ShapeDtypeStruct((NC, n_out, 16), jnp.float32),
        mesh=_mesh(),
        scratch_types=[
            pltpu.VMEM((CH, LANE), jnp.int32),
            pltpu.VMEM((CH, LANE), jnp.int32),
            pltpu.VMEM((CH * LANE, 16), jnp.float32),
            pltpu.VMEM_SHARED((n_out, 16), jnp.float32),
            pltpu.SemaphoreType.DMA,
        ],
        compiler_params=_SC_PARAMS,
    )
    return call(g2, srcx, dstr, zeros_hbm)


def _tc_prep_idx(src_r, n_nodes):
    rt = src_r.shape[0]
    blk = rt // 8

    def body(s_ref, o_ref):
        v = s_ref[...]
        o_ref[0] = v
        o_ref[1] = v + n_nodes

    return pl.pallas_call(
        body,
        grid=(rt // blk,),
        in_specs=[pl.BlockSpec((blk, LANE), lambda i: (i, 0))],
        out_specs=pl.BlockSpec((2, blk, LANE), lambda i: (0, i, 0)),
        out_shape=jax.ShapeDtypeStruct((2, rt, LANE), jnp.int32),
    )(src_r)


def _tc_encoder(x, W1, b1, W2, b2, Wc0, degp, blk):
    n = x.shape[0]

    def body(x_ref, w1_ref, b1_ref, w2_ref, b2_ref, wc_ref, d0_ref, d1_ref,
             dinv_ref, g_ref):
        deg = d0_ref[0][:, 0:1] + d1_ref[0][:, 0:1] + 1.0
        dinv = lax.rsqrt(deg)
        h = jnp.maximum(jnp.dot(x_ref[...], w1_ref[...], precision=_HIGH)
                        + b1_ref[...], 0.0)
        h = jnp.dot(h, w2_ref[...], precision=_HIGH) + b2_ref[...]
        g = jnp.dot(h, wc_ref[...], precision=_HIGH) * dinv
        dinv_ref[...] = dinv
        g_ref[0] = g[:, :16]
        g_ref[1] = g[:, 16:]

    return pl.pallas_call(
        body,
        grid=(n // blk,),
        in_specs=[
            pl.BlockSpec((blk, 128), lambda i: (i, 0)),
            pl.BlockSpec((128, 32), lambda i: (0, 0)),
            pl.BlockSpec((1, 32), lambda i: (0, 0)),
            pl.BlockSpec((32, 32), lambda i: (0, 0)),
            pl.BlockSpec((1, 32), lambda i: (0, 0)),
            pl.BlockSpec((32, 32), lambda i: (0, 0)),
            pl.BlockSpec((1, blk, 16), lambda i: (0, i, 0)),
            pl.BlockSpec((1, blk, 16), lambda i: (1, i, 0)),
        ],
        out_specs=[
            pl.BlockSpec((blk, 1), lambda i: (i, 0)),
            pl.BlockSpec((2, blk, 16), lambda i: (0, i, 0)),
        ],
        out_shape=[
            jax.ShapeDtypeStruct((n, 1), jnp.float32),
            jax.ShapeDtypeStruct((2, n, 16), jnp.float32),
        ],
    )(x, W1, b1, W2, b2, Wc0, degp, degp)


def _tc_mid(acc, g, dinv, b, Wn, blk):
    n = dinv.shape[0]

    def body(a0, a1, g0, g1, d_ref, b_ref, w_ref, o_ref):
        accb = jnp.concatenate([a0[0], a1[0]], axis=1)
        gb = jnp.concatenate([g0[0], g1[0]], axis=1)
        dinv_b = d_ref[...]
        h = jnp.maximum(dinv_b * (accb + gb) + b_ref[...], 0.0)
        gn = jnp.dot(h, w_ref[...], precision=_HIGH) * dinv_b
        o_ref[0] = gn[:, :16]
        o_ref[1] = gn[:, 16:]

    return pl.pallas_call(
        body,
        grid=(n // blk,),
        in_specs=[
            pl.BlockSpec((1, blk, 16), lambda i: (0, i, 0)),
            pl.BlockSpec((1, blk, 16), lambda i: (1, i, 0)),
            pl.BlockSpec((1, blk, 16), lambda i: (0, i, 0)),
            pl.BlockSpec((1, blk, 16), lambda i: (1, i, 0)),
            pl.BlockSpec((blk, 1), lambda i: (i, 0)),
            pl.BlockSpec((1, 32), lambda i: (0, 0)),
            pl.BlockSpec((32, 32), lambda i: (0, 0)),
        ],
        out_specs=pl.BlockSpec((2, blk, 16), lambda i: (0, i, 0)),
        out_shape=jax.ShapeDtypeStruct((2, n, 16), jnp.float32),
    )(acc, acc, g, g, dinv, b, Wn)


def _tc_final(acc, g, dinv, b, blk):
    n = dinv.shape[0]

    def body(a0, a1, g0, g1, d_ref, b_ref, o_ref):
        accb = jnp.concatenate([a0[0], a1[0]], axis=1)
        gb = jnp.concatenate([g0[0], g1[0]], axis=1)
        o_ref[...] = jnp.maximum(d_ref[...] * (accb + gb) + b_ref[...], 0.0)

    return pl.pallas_call(
        body,
        grid=(n // blk,),
        in_specs=[
            pl.BlockSpec((1, blk, 16), lambda i: (0, i, 0)),
            pl.BlockSpec((1, blk, 16), lambda i: (1, i, 0)),
            pl.BlockSpec((1, blk, 16), lambda i: (0, i, 0)),
            pl.BlockSpec((1, blk, 16), lambda i: (1, i, 0)),
            pl.BlockSpec((blk, 1), lambda i: (i, 0)),
            pl.BlockSpec((1, 32), lambda i: (0, 0)),
        ],
        out_specs=pl.BlockSpec((blk, 32), lambda i: (i, 0)),
        out_shape=jax.ShapeDtypeStruct((n, 32), jnp.float32),
    )(acc, acc, g, g, dinv, b)


def kernel(x, edge_index, W1, b1, W2, b2, Wc0, bc0, Wc1, bc1, Wc2, bc2):
    n, f_in = x.shape
    e = edge_index.shape[1]
    assert f_in == 128 and W1.shape[1] == 32
    assert n % NS == 0

    rows = -(-e // LANE)
    rt = -(-rows // (NS * CH * 2)) * (NS * CH * 2)
    pad = rt * LANE - e
    src = jnp.concatenate([edge_index[0], jnp.zeros((pad,), jnp.int32)])
    dst = jnp.concatenate([edge_index[1], jnp.full((pad,), n, jnp.int32)])
    src_r = src.reshape(rt, LANE)
    dstr = dst.reshape(rt, LANE)
    srcx = _tc_prep_idx(src_r, n)

    n_out = -(-(n + 1) // (NS * 8)) * (NS * 8)
    zb = CH * LANE
    zeros_hbm = jnp.zeros((zb, 16), jnp.float32)
    ones_hbm = jnp.ones((LANE, 16), jnp.float32)

    degp = _sc_degree(dstr, ones_hbm, zeros_hbm, n_out, zb)

    blk = 2000
    b1r = b1.reshape(1, 32)
    b2r = b2.reshape(1, 32)
    dinv, g = _tc_encoder(x, W1, b1r, W2, b2r, Wc0, degp, blk)

    for (bc, Wn) in ((bc0, Wc1), (bc1, Wc2)):
        acc = _sc_layer(g.reshape(2 * n, 16), srcx, dstr, zeros_hbm, n_out)
        g = _tc_mid(acc, g, dinv, bc.reshape(1, 32), Wn, blk)

    acc = _sc_layer(g.reshape(2 * n, 16), srcx, dstr, zeros_hbm, n_out)
    return _tc_final(acc, g, dinv, bc2.reshape(1, 32), blk)


# speedup vs baseline: 20.0695x; 1.0110x over previous
import functools

import jax
import jax.numpy as jnp
from jax import lax
from jax.experimental import pallas as pl
from jax.experimental.pallas import tpu as pltpu
from jax.experimental.pallas import tpu_sc as plsc

NC = 2
NS = 16
LANE = 128
CH = 8

_HIGH = jax.lax.Precision.HIGHEST


def _mesh():
    return plsc.VectorSubcoreMesh(
        core_axis_name="c", subcore_axis_name="s", num_cores=NC, num_subcores=NS
    )


_SC_PARAMS = pltpu.CompilerParams(use_tc_tiling_on_sc=False)


def _sc_degree(dstr, ones_hbm, zeros_hbm, n_out, zb):
    rt = dstr.shape[0]
    rows_w = rt // (NC * NS)
    nch = rows_w // CH
    zr = n_out // NS
    out_r = n_out // NS
    zfull, zrem = zr // zb, zr % zb

    def body(dstr_h, ones_h, zeros_h, out_h, dst_v, ones_v, zbuf, deg_sh):
        c = lax.axis_index("c")
        s = lax.axis_index("s")
        pltpu.sync_copy(zeros_h, zbuf)
        pltpu.sync_copy(ones_h, ones_v)
        for z in range(zfull):
            pltpu.sync_copy(zbuf, deg_sh.at[pl.ds(s * zr + z * zb, zb)])
        if zrem:
            pltpu.sync_copy(zbuf.at[pl.ds(0, zrem)],
                            deg_sh.at[pl.ds(s * zr + zfull * zb, zrem)])
        plsc.subcore_barrier()
        base = (c * NS + s) * rows_w

        def chunk(i, _):
            pltpu.sync_copy(dstr_h.at[pl.ds(base + i * CH, CH)], dst_v)
            for j in range(CH):
                pltpu.sync_copy(ones_v, deg_sh.at[dst_v.at[j]], add=True)
            return ()

        lax.fori_loop(0, nch, chunk, ())
        plsc.subcore_barrier()
        pltpu.sync_copy(
            deg_sh.at[pl.ds(s * out_r, out_r)], out_h.at[c, pl.ds(s * out_r, out_r)]
        )

    call = pl.kernel(
        body,
        out_type=jax.ShapeDtypeStruct((NC, n_out, 16), jnp.float32),
        mesh=_mesh(),
        scratch_types=[
            pltpu.VMEM((CH, LANE), jnp.int32),
            pltpu.VMEM((LANE, 16), jnp.float32),
            pltpu.VMEM((zb, 16), jnp.float32),
            pltpu.VMEM_SHARED((n_out, 16), jnp.float32),
        ],
        compiler_params=_SC_PARAMS,
    )
    return call(dstr, ones_hbm, zeros_hbm)


NB = 4
CHL = 2


def _sc_layer(g2, srcx, dstr, zeros_hbm, n_out):
    rt = dstr.shape[0]
    rows_w = rt // NS
    nch = rows_w // CHL
    assert nch % NB == 0 and nch // NB >= 2
    zr = n_out // NS
    out_r = n_out // NS
    zb = CHL * LANE
    zfull, zrem = zr // zb, zr % zb

    def body(g2_h, srcx_h, dstr_h, zeros_h, out_h,
             src_v, dst_v, rows_v, acc_sh, *sems):
        gsem = sems[:NB]
        ssem = sems[NB:]
        c = lax.axis_index("c")
        s = lax.axis_index("s")
        pltpu.sync_copy(zeros_h, rows_v.at[0])
        for z in range(zfull):
            pltpu.sync_copy(rows_v.at[0], acc_sh.at[pl.ds(s * zr + z * zb, zb)])
        if zrem:
            pltpu.sync_copy(rows_v.at[0, pl.ds(0, zrem)],
                            acc_sh.at[pl.ds(s * zr + zfull * zb, zrem)])
        plsc.subcore_barrier()
        base = s * rows_w

        def load_fire(k, b):
            rb = base + k * CHL
            pltpu.sync_copy(srcx_h.at[c, pl.ds(rb, CHL)], src_v.at[b])
            pltpu.sync_copy(dstr_h.at[pl.ds(rb, CHL)], dst_v.at[b])
            for j in range(CHL):
                pltpu.async_copy(
                    g2_h.at[src_v.at[b, j]],
                    rows_v.at[b, pl.ds(j * LANE, LANE)],
                    gsem[b],
                )

        def wait_g(b):
            for j in range(CHL):
                pltpu.make_async_copy(
                    g2_h.at[src_v.at[b, j]],
                    rows_v.at[b, pl.ds(j * LANE, LANE)],
                    gsem[b],
                ).wait()

        def fire_scatter(b):
            for j in range(CHL):
                pltpu.async_copy(
                    rows_v.at[b, pl.ds(j * LANE, LANE)],
                    acc_sh.at[dst_v.at[b, j]],
                    ssem[b],
                    add=True,
                )

        def wait_s(b):
            for j in range(CHL):
                pltpu.make_async_copy(
                    rows_v.at[b, pl.ds(j * LANE, LANE)],
                    acc_sh.at[dst_v.at[b, j]],
                    ssem[b],
                ).wait()

        load_fire(0, 0)
        for b in range(1, NB):
            load_fire(b, b)
            wait_g(b - 1)
            fire_scatter(b - 1)

        def round_body(r, _):
            for b in range(NB):
                k = r * NB + b
                wait_s(b)
                load_fire(k, b)
                wait_g((b - 1) % NB)
                fire_scatter((b - 1) % NB)
            return ()

        lax.fori_loop(1, nch // NB, round_body, ())
        wait_g(NB - 1)
        fire_scatter(NB - 1)
        for b in range(NB):
            wait_s(b)
        plsc.subcore_barrier()
        pltpu.sync_copy(
            acc_sh.at[pl.ds(s * out_r, out_r)], out_h.at[c, pl.ds(s * out_r, out_r)]
        )

    call = pl.kernel(
        body,
        out_type=jax.ShapeDtypeStruct((NC, n_out, 16), jnp.float32),
        mesh=_mesh(),
        scratch_types=[
            pltpu.VMEM((NB, CHL, LANE), jnp.int32),
            pltpu.VMEM((NB, CHL, LANE), jnp.int32),
            pltpu.VMEM((NB, CHL * LANE, 16), jnp.float32),
            pltpu.VMEM_SHARED((n_out, 16), jnp.float32),
        ] + [pltpu.SemaphoreType.DMA] * (2 * NB),
        compiler_params=_SC_PARAMS,
    )
    return call(g2, srcx, dstr, zeros_hbm)


def _tc_prep_idx(src_r, n_nodes):
    rt = src_r.shape[0]
    blk = rt // 8

    def body(s_ref, o_ref):
        v = s_ref[...]
        o_ref[0] = v
        o_ref[1] = v + n_nodes

    return pl.pallas_call(
        body,
        grid=(rt // blk,),
        in_specs=[pl.BlockSpec((blk, LANE), lambda i: (i, 0))],
        out_specs=pl.BlockSpec((2, blk, LANE), lambda i: (0, i, 0)),
        out_shape=jax.ShapeDtypeStruct((2, rt, LANE), jnp.int32),
    )(src_r)


def _tc_encoder(x, W1, b1, W2, b2, Wc0, degp, blk):
    n = x.shape[0]

    def body(x_ref, w1_ref, b1_ref, w2_ref, b2_ref, wc_ref, d0_ref, d1_ref,
             dinv_ref, g_ref):
        deg = d0_ref[0][:, 0:1] + d1_ref[0][:, 0:1] + 1.0
        dinv = lax.rsqrt(deg)
        h = jnp.maximum(jnp.dot(x_ref[...], w1_ref[...], precision=_HIGH)
                        + b1_ref[...], 0.0)
        h = jnp.dot(h, w2_ref[...], precision=_HIGH) + b2_ref[...]
        g = jnp.dot(h, wc_ref[...], precision=_HIGH) * dinv
        dinv_ref[...] = dinv
        g_ref[0] = g[:, :16]
        g_ref[1] = g[:, 16:]

    return pl.pallas_call(
        body,
        grid=(n // blk,),
        in_specs=[
            pl.BlockSpec((blk, 128), lambda i: (i, 0)),
            pl.BlockSpec((128, 32), lambda i: (0, 0)),
            pl.BlockSpec((1, 32), lambda i: (0, 0)),
            pl.BlockSpec((32, 32), lambda i: (0, 0)),
            pl.BlockSpec((1, 32), lambda i: (0, 0)),
            pl.BlockSpec((32, 32), lambda i: (0, 0)),
            pl.BlockSpec((1, blk, 16), lambda i: (0, i, 0)),
            pl.BlockSpec((1, blk, 16), lambda i: (1, i, 0)),
        ],
        out_specs=[
            pl.BlockSpec((blk, 1), lambda i: (i, 0)),
            pl.BlockSpec((2, blk, 16), lambda i: (0, i, 0)),
        ],
        out_shape=[
            jax.ShapeDtypeStruct((n, 1), jnp.float32),
            jax.ShapeDtypeStruct((2, n, 16), jnp.float32),
        ],
    )(x, W1, b1, W2, b2, Wc0, degp, degp)


def _tc_mid(acc, g, dinv, b, Wn, blk):
    n = dinv.shape[0]

    def body(a0, a1, g0, g1, d_ref, b_ref, w_ref, o_ref):
        accb = jnp.concatenate([a0[0], a1[0]], axis=1)
        gb = jnp.concatenate([g0[0], g1[0]], axis=1)
        dinv_b = d_ref[...]
        h = jnp.maximum(dinv_b * (accb + gb) + b_ref[...], 0.0)
        gn = jnp.dot(h, w_ref[...], precision=_HIGH) * dinv_b
        o_ref[0] = gn[:, :16]
        o_ref[1] = gn[:, 16:]

    return pl.pallas_call(
        body,
        grid=(n // blk,),
        in_specs=[
            pl.BlockSpec((1, blk, 16), lambda i: (0, i, 0)),
            pl.BlockSpec((1, blk, 16), lambda i: (1, i, 0)),
            pl.BlockSpec((1, blk, 16), lambda i: (0, i, 0)),
            pl.BlockSpec((1, blk, 16), lambda i: (1, i, 0)),
            pl.BlockSpec((blk, 1), lambda i: (i, 0)),
            pl.BlockSpec((1, 32), lambda i: (0, 0)),
            pl.BlockSpec((32, 32), lambda i: (0, 0)),
        ],
        out_specs=pl.BlockSpec((2, blk, 16), lambda i: (0, i, 0)),
        out_shape=jax.ShapeDtypeStruct((2, n, 16), jnp.float32),
    )(acc, acc, g, g, dinv, b, Wn)


def _tc_final(acc, g, dinv, b, blk):
    n = dinv.shape[0]

    def body(a0, a1, g0, g1, d_ref, b_ref, o_ref):
        accb = jnp.concatenate([a0[0], a1[0]], axis=1)
        gb = jnp.concatenate([g0[0], g1[0]], axis=1)
        o_ref[...] = jnp.maximum(d_ref[...] * (accb + gb) + b_ref[...], 0.0)

    return pl.pallas_call(
        body,
        grid=(n // blk,),
        in_specs=[
            pl.BlockSpec((1, blk, 16), lambda i: (0, i, 0)),
            pl.BlockSpec((1, blk, 16), lambda i: (1, i, 0)),
            pl.BlockSpec((1, blk, 16), lambda i: (0, i, 0)),
            pl.BlockSpec((1, blk, 16), lambda i: (1, i, 0)),
            pl.BlockSpec((blk, 1), lambda i: (i, 0)),
            pl.BlockSpec((1, 32), lambda i: (0, 0)),
        ],
        out_specs=pl.BlockSpec((blk, 32), lambda i: (i, 0)),
        out_shape=jax.ShapeDtypeStruct((n, 32), jnp.float32),
    )(acc, acc, g, g, dinv, b)


def kernel(x, edge_index, W1, b1, W2, b2, Wc0, bc0, Wc1, bc1, Wc2, bc2):
    n, f_in = x.shape
    e = edge_index.shape[1]
    assert f_in == 128 and W1.shape[1] == 32
    assert n % NS == 0

    rows = -(-e // LANE)
    rt = -(-rows // (NS * CH * 2)) * (NS * CH * 2)
    pad = rt * LANE - e
    src = jnp.concatenate([edge_index[0], jnp.zeros((pad,), jnp.int32)])
    dst = jnp.concatenate([edge_index[1], jnp.full((pad,), n, jnp.int32)])
    src_r = src.reshape(rt, LANE)
    dstr = dst.reshape(rt, LANE)
    srcx = _tc_prep_idx(src_r, n)

    n_out = -(-(n + 1) // (NS * 8)) * (NS * 8)
    zb = CHL * LANE
    zeros_hbm = jnp.zeros((zb, 16), jnp.float32)
    ones_hbm = jnp.ones((LANE, 16), jnp.float32)

    degp = _sc_degree(dstr, ones_hbm, zeros_hbm, n_out, zb)

    blk = 2000
    b1r = b1.reshape(1, 32)
    b2r = b2.reshape(1, 32)
    dinv, g = _tc_encoder(x, W1, b1r, W2, b2r, Wc0, degp, blk)

    for (bc, Wn) in ((bc0, Wc1), (bc1, Wc2)):
        acc = _sc_layer(g.reshape(2 * n, 16), srcx, dstr, zeros_hbm, n_out)
        g = _tc_mid(acc, g, dinv, bc.reshape(1, 32), Wn, blk)

    acc = _sc_layer(g.reshape(2 * n, 16), srcx, dstr, zeros_hbm, n_out)
    return _tc_final(acc, g, dinv, bc2.reshape(1, 32), blk)


# speedup vs baseline: 23.4991x; 1.1709x over previous
import functools

import jax
import jax.numpy as jnp
from jax import lax
from jax.experimental import pallas as pl
from jax.experimental.pallas import tpu as pltpu
from jax.experimental.pallas import tpu_sc as plsc

NC = 2
NS = 16
LANE = 128
CH = 6

_HIGH = jax.lax.Precision.HIGHEST


def _mesh():
    return plsc.VectorSubcoreMesh(
        core_axis_name="c", subcore_axis_name="s", num_cores=NC, num_subcores=NS
    )


_SC_PARAMS = pltpu.CompilerParams(use_tc_tiling_on_sc=False)


def _sc_degree(dstr, ones_hbm, zeros_hbm, n_out, zb):
    rt = dstr.shape[0]
    rows_w = rt // (NC * NS)
    nch = rows_w // CH
    zr = n_out // NS
    out_r = n_out // NS
    zfull, zrem = zr // zb, zr % zb

    def body(dstr_h, ones_h, zeros_h, out_h, dst_v, ones_v, zbuf, deg_sh):
        c = lax.axis_index("c")
        s = lax.axis_index("s")
        pltpu.sync_copy(zeros_h, zbuf)
        pltpu.sync_copy(ones_h, ones_v)
        for z in range(zfull):
            pltpu.sync_copy(zbuf, deg_sh.at[pl.ds(s * zr + z * zb, zb)])
        if zrem:
            pltpu.sync_copy(zbuf.at[pl.ds(0, zrem)],
                            deg_sh.at[pl.ds(s * zr + zfull * zb, zrem)])
        plsc.subcore_barrier()
        base = (c * NS + s) * rows_w

        def chunk(i, _):
            pltpu.sync_copy(dstr_h.at[pl.ds(base + i * CH, CH)], dst_v)
            for j in range(CH):
                pltpu.sync_copy(ones_v, deg_sh.at[dst_v.at[j]], add=True)
            return ()

        lax.fori_loop(0, nch, chunk, ())
        plsc.subcore_barrier()
        pltpu.sync_copy(
            deg_sh.at[pl.ds(s * out_r, out_r)], out_h.at[c, pl.ds(s * out_r, out_r)]
        )

    call = pl.kernel(
        body,
        out_type=jax.ShapeDtypeStruct((NC, n_out, 16), jnp.float32),
        mesh=_mesh(),
        scratch_types=[
            pltpu.VMEM((CH, LANE), jnp.int32),
            pltpu.VMEM((LANE, 16), jnp.float32),
            pltpu.VMEM((zb, 16), jnp.float32),
            pltpu.VMEM_SHARED((n_out, 16), jnp.float32),
        ],
        compiler_params=_SC_PARAMS,
    )
    return call(dstr, ones_hbm, zeros_hbm)


CHL = 6


def _sc_layer(g2, srcx_f, dstr_f, zeros_hbm, n_out, rt):
    rows_w = rt // NS
    nch = rows_w // CHL
    assert nch % 4 == 0 and nch >= 8
    ew = CHL * LANE
    zr = n_out // NS
    out_r = n_out // NS
    zb = ew
    zfull, zrem = zr // zb, zr % zb

    def body(g2_h, srcx_h, dstr_h, zeros_h, out_h,
             src_v, dst_v, rows_v, acc_sh, *sems):
        gsem = sems[0:2]
        ssem = sems[2:4]
        isrc = sems[4:6]
        idst = sems[6:10]
        c = lax.axis_index("c")
        s = lax.axis_index("s")
        pltpu.sync_copy(zeros_h, rows_v.at[0])
        for z in range(zfull):
            pltpu.sync_copy(rows_v.at[0], acc_sh.at[pl.ds(s * zr + z * zb, zb)])
        if zrem:
            pltpu.sync_copy(rows_v.at[0, pl.ds(0, zrem)],
                            acc_sh.at[pl.ds(s * zr + zfull * zb, zrem)])
        plsc.subcore_barrier()
        ebase = s * rows_w * LANE

        def load_src(k, slot, sync=False):
            sl = srcx_h.at[c, pl.ds(ebase + k * ew, ew)]
            if sync:
                pltpu.sync_copy(sl, src_v.at[slot])
            else:
                pltpu.async_copy(sl, src_v.at[slot], isrc[slot])

        def load_dst(k, slot, sync=False):
            sl = dstr_h.at[pl.ds(ebase + k * ew, ew)]
            if sync:
                pltpu.sync_copy(sl, dst_v.at[slot])
            else:
                pltpu.async_copy(sl, dst_v.at[slot], idst[slot])

        def wait_isrc(slot):
            pltpu.make_async_copy(
                srcx_h.at[c, pl.ds(0, ew)], src_v.at[slot], isrc[slot]).wait()

        def wait_idst(slot):
            pltpu.make_async_copy(
                dstr_h.at[pl.ds(0, ew)], dst_v.at[slot], idst[slot]).wait()

        def fire_g(b):
            for j in range(CHL):
                pltpu.async_copy(
                    g2_h.at[src_v.at[b, pl.ds(j * LANE, LANE)]],
                    rows_v.at[b, pl.ds(j * LANE, LANE)], gsem[b])

        def wait_g(b):
            for j in range(CHL):
                pltpu.make_async_copy(
                    g2_h.at[src_v.at[b, pl.ds(j * LANE, LANE)]],
                    rows_v.at[b, pl.ds(j * LANE, LANE)], gsem[b]).wait()

        def fire_s(b, d):
            for j in range(CHL):
                pltpu.async_copy(
                    rows_v.at[b, pl.ds(j * LANE, LANE)],
                    acc_sh.at[dst_v.at[d, pl.ds(j * LANE, LANE)]],
                    ssem[b], add=True)

        def wait_s(b, d):
            for j in range(CHL):
                pltpu.make_async_copy(
                    rows_v.at[b, pl.ds(j * LANE, LANE)],
                    acc_sh.at[dst_v.at[d, pl.ds(j * LANE, LANE)]],
                    ssem[b]).wait()

        def visit_steady(k, b, b1, d, d2, first=False, second=False):
            if not first:
                wait_s(b1, (d + 3) % 4)
            if not first:
                wait_isrc(b1)
            fire_g(b1)
            wait_g(b)
            load_src(k + 2, b)
            if not (first or second):
                wait_idst(d)
            load_dst(k + 2, d2)
            fire_s(b, d)

        load_src(0, 0, sync=True)
        load_src(1, 1, sync=True)
        load_dst(0, 0, sync=True)
        load_dst(1, 1, sync=True)
        fire_g(0)
        visit_steady(0, 0, 1, 0, 2, first=True)
        visit_steady(1, 1, 0, 1, 3, second=True)
        visit_steady(2, 0, 1, 2, 0)
        visit_steady(3, 1, 0, 3, 1)

        def round_body(r, _):
            k = r * 4
            visit_steady(k + 0, 0, 1, 0, 2)
            visit_steady(k + 1, 1, 0, 1, 3)
            visit_steady(k + 2, 0, 1, 2, 0)
            visit_steady(k + 3, 1, 0, 3, 1)
            return ()

        lax.fori_loop(1, nch // 4, round_body, ())
        wait_g(0)
        wait_s(1, 3)
        wait_isrc(1)
        wait_idst(0)
        wait_idst(1)
        plsc.subcore_barrier()
        pltpu.sync_copy(
            acc_sh.at[pl.ds(s * out_r, out_r)], out_h.at[c, pl.ds(s * out_r, out_r)]
        )

    call = pl.kernel(
        body,
        out_type=jax.ShapeDtypeStruct((NC, n_out, 16), jnp.float32),
        mesh=_mesh(),
        scratch_types=[
            pltpu.VMEM((2, CHL * LANE), jnp.int32),
            pltpu.VMEM((4, CHL * LANE), jnp.int32),
            pltpu.VMEM((2, CHL * LANE, 16), jnp.float32),
            pltpu.VMEM_SHARED((n_out, 16), jnp.float32),
        ] + [pltpu.SemaphoreType.DMA] * 10,
        compiler_params=_SC_PARAMS,
    )
    return call(g2, srcx_f, dstr_f, zeros_hbm)


def _tc_prep_idx(src_r, n_nodes):
    rt = src_r.shape[0]
    blk = 8
    for d in range(2048, 7, -8):
        if rt % d == 0:
            blk = d
            break

    def body(s_ref, o_ref):
        v = s_ref[...]
        o_ref[0] = v
        o_ref[1] = v + n_nodes

    return pl.pallas_call(
        body,
        grid=(rt // blk,),
        in_specs=[pl.BlockSpec((blk, LANE), lambda i: (i, 0))],
        out_specs=pl.BlockSpec((2, blk, LANE), lambda i: (0, i, 0)),
        out_shape=jax.ShapeDtypeStruct((2, rt, LANE), jnp.int32),
    )(src_r)


def _tc_encoder(x, W1, b1, W2, b2, Wc0, degp, blk):
    n = x.shape[0]

    def body(x_ref, w1_ref, b1_ref, w2_ref, b2_ref, wc_ref, d0_ref, d1_ref,
             dinv_ref, g_ref):
        deg = d0_ref[0][:, 0:1] + d1_ref[0][:, 0:1] + 1.0
        dinv = lax.rsqrt(deg)
        h = jnp.maximum(jnp.dot(x_ref[...], w1_ref[...], precision=_HIGH)
                        + b1_ref[...], 0.0)
        h = jnp.dot(h, w2_ref[...], precision=_HIGH) + b2_ref[...]
        g = jnp.dot(h, wc_ref[...], precision=_HIGH) * dinv
        dinv_ref[...] = dinv
        g_ref[0] = g[:, :16]
        g_ref[1] = g[:, 16:]

    return pl.pallas_call(
        body,
        grid=(n // blk,),
        in_specs=[
            pl.BlockSpec((blk, 128), lambda i: (i, 0)),
            pl.BlockSpec((128, 32), lambda i: (0, 0)),
            pl.BlockSpec((1, 32), lambda i: (0, 0)),
            pl.BlockSpec((32, 32), lambda i: (0, 0)),
            pl.BlockSpec((1, 32), lambda i: (0, 0)),
            pl.BlockSpec((32, 32), lambda i: (0, 0)),
            pl.BlockSpec((1, blk, 16), lambda i: (0, i, 0)),
            pl.BlockSpec((1, blk, 16), lambda i: (1, i, 0)),
        ],
        out_specs=[
            pl.BlockSpec((blk, 1), lambda i: (i, 0)),
            pl.BlockSpec((2, blk, 16), lambda i: (0, i, 0)),
        ],
        out_shape=[
            jax.ShapeDtypeStruct((n, 1), jnp.float32),
            jax.ShapeDtypeStruct((2, n, 16), jnp.float32),
        ],
    )(x, W1, b1, W2, b2, Wc0, degp, degp)


def _tc_mid(acc, g, dinv, b, Wn, blk):
    n = dinv.shape[0]

    def body(a0, a1, g0, g1, d_ref, b_ref, w_ref, o_ref):
        accb = jnp.concatenate([a0[0], a1[0]], axis=1)
        gb = jnp.concatenate([g0[0], g1[0]], axis=1)
        dinv_b = d_ref[...]
        h = jnp.maximum(dinv_b * (accb + gb) + b_ref[...], 0.0)
        gn = jnp.dot(h, w_ref[...], precision=_HIGH) * dinv_b
        o_ref[0] = gn[:, :16]
        o_ref[1] = gn[:, 16:]

    return pl.pallas_call(
        body,
        grid=(n // blk,),
        in_specs=[
            pl.BlockSpec((1, blk, 16), lambda i: (0, i, 0)),
            pl.BlockSpec((1, blk, 16), lambda i: (1, i, 0)),
            pl.BlockSpec((1, blk, 16), lambda i: (0, i, 0)),
            pl.BlockSpec((1, blk, 16), lambda i: (1, i, 0)),
            pl.BlockSpec((blk, 1), lambda i: (i, 0)),
            pl.BlockSpec((1, 32), lambda i: (0, 0)),
            pl.BlockSpec((32, 32), lambda i: (0, 0)),
        ],
        out_specs=pl.BlockSpec((2, blk, 16), lambda i: (0, i, 0)),
        out_shape=jax.ShapeDtypeStruct((2, n, 16), jnp.float32),
    )(acc, acc, g, g, dinv, b, Wn)


def _tc_final(acc, g, dinv, b, blk):
    n = dinv.shape[0]

    def body(a0, a1, g0, g1, d_ref, b_ref, o_ref):
        accb = jnp.concatenate([a0[0], a1[0]], axis=1)
        gb = jnp.concatenate([g0[0], g1[0]], axis=1)
        o_ref[...] = jnp.maximum(d_ref[...] * (accb + gb) + b_ref[...], 0.0)

    return pl.pallas_call(
        body,
        grid=(n // blk,),
        in_specs=[
            pl.BlockSpec((1, blk, 16), lambda i: (0, i, 0)),
            pl.BlockSpec((1, blk, 16), lambda i: (1, i, 0)),
            pl.BlockSpec((1, blk, 16), lambda i: (0, i, 0)),
            pl.BlockSpec((1, blk, 16), lambda i: (1, i, 0)),
            pl.BlockSpec((blk, 1), lambda i: (i, 0)),
            pl.BlockSpec((1, 32), lambda i: (0, 0)),
        ],
        out_specs=pl.BlockSpec((blk, 32), lambda i: (i, 0)),
        out_shape=jax.ShapeDtypeStruct((n, 32), jnp.float32),
    )(acc, acc, g, g, dinv, b)


def kernel(x, edge_index, W1, b1, W2, b2, Wc0, bc0, Wc1, bc1, Wc2, bc2):
    n, f_in = x.shape
    e = edge_index.shape[1]
    assert f_in == 128 and W1.shape[1] == 32
    assert n % NS == 0

    rows = -(-e // LANE)
    rt = -(-rows // (NS * CHL * 4)) * (NS * CHL * 4)
    assert (rt // (NC * NS)) % CH == 0
    rtp = rt + 16
    pad = rtp * LANE - e
    src = jnp.concatenate([edge_index[0], jnp.zeros((pad,), jnp.int32)])
    dst = jnp.concatenate([edge_index[1], jnp.full((pad,), n, jnp.int32)])
    src_r = src.reshape(rtp, LANE)
    dstr = dst[: rt * LANE].reshape(rt, LANE)
    srcx = _tc_prep_idx(src_r, n)

    n_out = -(-(n + 1) // (NS * 8)) * (NS * 8)
    zb = CHL * LANE
    zeros_hbm = jnp.zeros((zb, 16), jnp.float32)
    ones_hbm = jnp.ones((LANE, 16), jnp.float32)

    degp = _sc_degree(dstr, ones_hbm, zeros_hbm, n_out, zb)

    blk = 2000
    b1r = b1.reshape(1, 32)
    b2r = b2.reshape(1, 32)
    dinv, g = _tc_encoder(x, W1, b1r, W2, b2r, Wc0, degp, blk)

    srcx_f = srcx.reshape(2, rtp * LANE)
    dstr_f = dst
    for (bc, Wn) in ((bc0, Wc1), (bc1, Wc2)):
        acc = _sc_layer(g.reshape(2 * n, 16), srcx_f, dstr_f, zeros_hbm, n_out, rt)
        g = _tc_mid(acc, g, dinv, bc.reshape(1, 32), Wn, blk)

    acc = _sc_layer(g.reshape(2 * n, 16), srcx_f, dstr_f, zeros_hbm, n_out, rt)
    return _tc_final(acc, g, dinv, bc2.reshape(1, 32), blk)


# speedup vs baseline: 30.9921x; 1.3189x over previous
import functools

import jax
import jax.numpy as jnp
from jax import lax
from jax.experimental import pallas as pl
from jax.experimental.pallas import tpu as pltpu
from jax.experimental.pallas import tpu_sc as plsc

NC = 2
NS = 16
LANE = 128
CH = 6

_HIGH = jax.lax.Precision.HIGHEST


def _mesh():
    return plsc.VectorSubcoreMesh(
        core_axis_name="c", subcore_axis_name="s", num_cores=NC, num_subcores=NS
    )


_SC_PARAMS = pltpu.CompilerParams(use_tc_tiling_on_sc=False)


def _sc_degree(dstr, ones_hbm, zeros_hbm, n_out, zb):
    rt = dstr.shape[0]
    rows_w = rt // (NC * NS)
    nch = rows_w // CH
    zr = n_out // NS
    out_r = n_out // NS
    zfull, zrem = zr // zb, zr % zb

    def body(dstr_h, ones_h, zeros_h, out_h, dst_v, ones_v, zbuf, deg_sh):
        c = lax.axis_index("c")
        s = lax.axis_index("s")
        pltpu.sync_copy(zeros_h, zbuf)
        pltpu.sync_copy(ones_h, ones_v)
        for z in range(zfull):
            pltpu.sync_copy(zbuf, deg_sh.at[pl.ds(s * zr + z * zb, zb)])
        if zrem:
            pltpu.sync_copy(zbuf.at[pl.ds(0, zrem)],
                            deg_sh.at[pl.ds(s * zr + zfull * zb, zrem)])
        plsc.subcore_barrier()
        base = (c * NS + s) * rows_w

        def chunk(i, _):
            pltpu.sync_copy(dstr_h.at[pl.ds(base + i * CH, CH)], dst_v)
            for j in range(CH):
                pltpu.sync_copy(ones_v, deg_sh.at[dst_v.at[j]], add=True)
            return ()

        lax.fori_loop(0, nch, chunk, ())
        plsc.subcore_barrier()
        pltpu.sync_copy(
            deg_sh.at[pl.ds(s * out_r, out_r)], out_h.at[c, pl.ds(s * out_r, out_r)]
        )

    call = pl.kernel(
        body,
        out_type=jax.ShapeDtypeStruct((NC, n_out, 16), jnp.float32),
        mesh=_mesh(),
        scratch_types=[
            pltpu.VMEM((CH, LANE), jnp.int32),
            pltpu.VMEM((LANE, 16), jnp.float32),
            pltpu.VMEM((zb, 16), jnp.float32),
            pltpu.VMEM_SHARED((n_out, 16), jnp.float32),
        ],
        compiler_params=_SC_PARAMS,
    )
    return call(dstr, ones_hbm, zeros_hbm)


CHL = 6


def _sc_layer(g2, srcx_f, dstr_f, zeros_hbm, n_out, rt):
    rows_w = rt // NS
    nch = rows_w // CHL
    assert nch % 4 == 0 and nch >= 8
    ew = CHL * LANE
    zr = n_out // NS
    out_r = n_out // NS
    zb = ew
    zfull, zrem = zr // zb, zr % zb

    def body(g2_h, srcx_h, dstr_h, zeros_h, out_h,
             src_v, dst_v, rows_v, acc_sh, *sems):
        gsem = sems[0:2]
        ssem = sems[2:4]
        isrc = sems[4:6]
        idst = sems[6:10]
        c = lax.axis_index("c")
        s = lax.axis_index("s")
        pltpu.sync_copy(zeros_h, rows_v.at[0])
        for z in range(zfull):
            pltpu.sync_copy(rows_v.at[0], acc_sh.at[pl.ds(s * zr + z * zb, zb)])
        if zrem:
            pltpu.sync_copy(rows_v.at[0, pl.ds(0, zrem)],
                            acc_sh.at[pl.ds(s * zr + zfull * zb, zrem)])
        plsc.subcore_barrier()
        ebase = s * rows_w * LANE

        def load_src(k, slot, sync=False):
            sl = srcx_h.at[c, pl.ds(ebase + k * ew, ew)]
            if sync:
                pltpu.sync_copy(sl, src_v.at[slot])
            else:
                pltpu.async_copy(sl, src_v.at[slot], isrc[slot])

        def load_dst(k, slot, sync=False):
            sl = dstr_h.at[pl.ds(ebase + k * ew, ew)]
            if sync:
                pltpu.sync_copy(sl, dst_v.at[slot])
            else:
                pltpu.async_copy(sl, dst_v.at[slot], idst[slot])

        def wait_isrc(slot):
            pltpu.make_async_copy(
                srcx_h.at[c, pl.ds(0, ew)], src_v.at[slot], isrc[slot]).wait()

        def wait_idst(slot):
            pltpu.make_async_copy(
                dstr_h.at[pl.ds(0, ew)], dst_v.at[slot], idst[slot]).wait()

        def fire_g(b):
            for j in range(CHL):
                pltpu.async_copy(
                    g2_h.at[src_v.at[b, pl.ds(j * LANE, LANE)]],
                    rows_v.at[b, pl.ds(j * LANE, LANE)], gsem[b])

        def wait_g(b):
            for j in range(CHL):
                pltpu.make_async_copy(
                    g2_h.at[src_v.at[b, pl.ds(j * LANE, LANE)]],
                    rows_v.at[b, pl.ds(j * LANE, LANE)], gsem[b]).wait()

        def fire_s(b, d):
            for j in range(CHL):
                pltpu.async_copy(
                    rows_v.at[b, pl.ds(j * LANE, LANE)],
                    acc_sh.at[dst_v.at[d, pl.ds(j * LANE, LANE)]],
                    ssem[b], add=True)

        def wait_s(b, d):
            for j in range(CHL):
                pltpu.make_async_copy(
                    rows_v.at[b, pl.ds(j * LANE, LANE)],
                    acc_sh.at[dst_v.at[d, pl.ds(j * LANE, LANE)]],
                    ssem[b]).wait()

        def visit_steady(k, b, b1, d, d2, first=False, second=False):
            if not first:
                wait_s(b1, (d + 3) % 4)
            if not first:
                wait_isrc(b1)
            fire_g(b1)
            wait_g(b)
            load_src(k + 2, b)
            if not (first or second):
                wait_idst(d)
            load_dst(k + 2, d2)
            fire_s(b, d)

        load_src(0, 0, sync=True)
        load_src(1, 1, sync=True)
        load_dst(0, 0, sync=True)
        load_dst(1, 1, sync=True)
        fire_g(0)
        visit_steady(0, 0, 1, 0, 2, first=True)
        visit_steady(1, 1, 0, 1, 3, second=True)
        visit_steady(2, 0, 1, 2, 0)
        visit_steady(3, 1, 0, 3, 1)

        def round_body(r, _):
            k = r * 4
            visit_steady(k + 0, 0, 1, 0, 2)
            visit_steady(k + 1, 1, 0, 1, 3)
            visit_steady(k + 2, 0, 1, 2, 0)
            visit_steady(k + 3, 1, 0, 3, 1)
            return ()

        lax.fori_loop(1, nch // 4, round_body, ())
        wait_g(0)
        wait_s(1, 3)
        wait_isrc(1)
        wait_idst(0)
        wait_idst(1)
        plsc.subcore_barrier()
        pltpu.sync_copy(
            acc_sh.at[pl.ds(s * out_r, out_r)], out_h.at[c, pl.ds(s * out_r, out_r)]
        )

    call = pl.kernel(
        body,
        out_type=jax.ShapeDtypeStruct((NC, n_out, 16), jnp.float32),
        mesh=_mesh(),
        scratch_types=[
            pltpu.VMEM((2, CHL * LANE), jnp.int32),
            pltpu.VMEM((4, CHL * LANE), jnp.int32),
            pltpu.VMEM((2, CHL * LANE, 16), jnp.float32),
            pltpu.VMEM_SHARED((n_out, 16), jnp.float32),
        ] + [pltpu.SemaphoreType.DMA] * 10,
        compiler_params=_SC_PARAMS,
    )
    return call(g2, srcx_f, dstr_f, zeros_hbm)


def _tc_prep_idx(src_r, n_nodes):
    rt = src_r.shape[0]
    blk = 8
    for d in range(2048, 7, -8):
        if rt % d == 0:
            blk = d
            break

    def body(s_ref, o_ref):
        v = s_ref[...]
        o_ref[0] = v
        o_ref[1] = v + n_nodes

    return pl.pallas_call(
        body,
        grid=(rt // blk,),
        in_specs=[pl.BlockSpec((blk, LANE), lambda i: (i, 0))],
        out_specs=pl.BlockSpec((2, blk, LANE), lambda i: (0, i, 0)),
        out_shape=jax.ShapeDtypeStruct((2, rt, LANE), jnp.int32),
    )(src_r)


def _tc_encoder(x3, W1, b1, W2, b2, Wc0, degp_pk, pr):
    npk = degp_pk.shape[1]

    def body(x3_ref, w1_ref, b1_ref, w2_ref, b2_ref, wc_ref, d0_ref, d1_ref,
             dinv_ref, g_ref):
        dinv = lax.rsqrt(d0_ref[0] + d1_ref[0] + 1.0)
        c0, c1 = [], []
        for a in range(8):
            h = jnp.maximum(
                jnp.dot(x3_ref[:, a, :], w1_ref[...], precision=_HIGH)
                + b1_ref[...], 0.0)
            h = jnp.dot(h, w2_ref[...], precision=_HIGH) + b2_ref[...]
            gw = jnp.dot(h, wc_ref[...], precision=_HIGH)
            c0.append(gw[:, :16])
            c1.append(gw[:, 16:])
        dinv_ref[...] = dinv
        g_ref[0] = jnp.concatenate(c0, axis=1) * dinv
        g_ref[1] = jnp.concatenate(c1, axis=1) * dinv

    return pl.pallas_call(
        body,
        grid=(npk // pr,),
        in_specs=[
            pl.BlockSpec((pr, 8, 128), lambda i: (i, 0, 0)),
            pl.BlockSpec((128, 32), lambda i: (0, 0)),
            pl.BlockSpec((1, 32), lambda i: (0, 0)),
            pl.BlockSpec((32, 32), lambda i: (0, 0)),
            pl.BlockSpec((1, 32), lambda i: (0, 0)),
            pl.BlockSpec((32, 32), lambda i: (0, 0)),
            pl.BlockSpec((1, pr, 128), lambda i: (0, i, 0)),
            pl.BlockSpec((1, pr, 128), lambda i: (1, i, 0)),
        ],
        out_specs=[
            pl.BlockSpec((pr, 128), lambda i: (i, 0)),
            pl.BlockSpec((2, pr, 128), lambda i: (0, i, 0)),
        ],
        out_shape=[
            jax.ShapeDtypeStruct((npk, 128), jnp.float32),
            jax.ShapeDtypeStruct((2, npk, 128), jnp.float32),
        ],
    )(x3, W1, b1, W2, b2, Wc0, degp_pk, degp_pk)


def _tc_mid(acc_pk, g_pk, dinv_pk, b_pk, Wd, pr):
    npk = dinv_pk.shape[0]
    n_steps = npk // pr

    def body(a0, a1, g0, g1, d_ref, b_ref, wd_ref, o_ref):
        dinv = d_ref[...]
        h0 = jnp.maximum(dinv * (a0[0] + g0[0]) + b_ref[0, 0:1, :], 0.0)
        h1 = jnp.maximum(dinv * (a1[0] + g1[0]) + b_ref[0, 1:2, :], 0.0)
        for v in range(2):
            gn = (jnp.dot(h0, wd_ref[0, 0, v], precision=_HIGH)
                  + jnp.dot(h1, wd_ref[0, 1, v], precision=_HIGH)) * dinv
            o_ref[v] = gn

    return pl.pallas_call(
        body,
        grid=(n_steps,),
        in_specs=[
            pl.BlockSpec((1, pr, 128), lambda i: (0, i, 0)),
            pl.BlockSpec((1, pr, 128), lambda i: (1, i, 0)),
            pl.BlockSpec((1, pr, 128), lambda i: (0, i, 0)),
            pl.BlockSpec((1, pr, 128), lambda i: (1, i, 0)),
            pl.BlockSpec((pr, 128), lambda i: (i, 0)),
            pl.BlockSpec((1, 2, 128), lambda i: (0, 0, 0)),
            pl.BlockSpec((1, 2, 2, 128, 128), lambda i: (0, 0, 0, 0, 0)),
        ],
        out_specs=pl.BlockSpec((2, pr, 128), lambda i: (0, i, 0)),
        out_shape=jax.ShapeDtypeStruct((2, npk, 128), jnp.float32),
    )(acc_pk, acc_pk, g_pk, g_pk, dinv_pk, b_pk, Wd)


def _tc_final(acc_pk, g_pk, dinv_pk, b_pk, pr):
    npk = acc_pk.shape[1]

    def body(a0, a1, g0, g1, d_ref, b_ref, o_ref):
        dinv = d_ref[...]
        o_ref[0] = jnp.maximum(dinv * (a0[0] + g0[0]) + b_ref[0, 0:1, :], 0.0)
        o_ref[1] = jnp.maximum(dinv * (a1[0] + g1[0]) + b_ref[0, 1:2, :], 0.0)

    return pl.pallas_call(
        body,
        grid=(npk // pr,),
        in_specs=[
            pl.BlockSpec((1, pr, 128), lambda i: (0, i, 0)),
            pl.BlockSpec((1, pr, 128), lambda i: (1, i, 0)),
            pl.BlockSpec((1, pr, 128), lambda i: (0, i, 0)),
            pl.BlockSpec((1, pr, 128), lambda i: (1, i, 0)),
            pl.BlockSpec((pr, 128), lambda i: (i, 0)),
            pl.BlockSpec((1, 2, 128), lambda i: (0, 0, 0)),
        ],
        out_specs=pl.BlockSpec((2, pr, 128), lambda i: (0, i, 0)),
        out_shape=jax.ShapeDtypeStruct((2, npk, 128), jnp.float32),
    )(acc_pk, acc_pk, g_pk, g_pk, dinv_pk, b_pk)


def kernel(x, edge_index, W1, b1, W2, b2, Wc0, bc0, Wc1, bc1, Wc2, bc2):
    n, f_in = x.shape
    e = edge_index.shape[1]
    assert f_in == 128 and W1.shape[1] == 32
    assert n % NS == 0

    rows = -(-e // LANE)
    rt = -(-rows // (NS * CHL * 4)) * (NS * CHL * 4)
    assert (rt // (NC * NS)) % CH == 0
    rtp = rt + 16
    pad = rtp * LANE - e

    n_out = -(-(n + 1) // (NS * 8)) * (NS * 8)
    npk = n_out // 8

    src = jnp.concatenate([edge_index[0], jnp.zeros((pad,), jnp.int32)])
    dst = jnp.concatenate([edge_index[1], jnp.full((pad,), n, jnp.int32)])
    src_r = src.reshape(rtp, LANE)
    dstr = dst[: rt * LANE].reshape(rt, LANE)
    srcx = _tc_prep_idx(src_r, n_out)
    srcx_f = srcx.reshape(2, rtp * LANE)
    dstr_f = dst

    zb = CHL * LANE
    zeros_hbm = jnp.zeros((zb, 16), jnp.float32)
    ones_hbm = jnp.ones((LANE, 16), jnp.float32)

    degp = _sc_degree(dstr, ones_hbm, zeros_hbm, n_out, zb)
    degp_pk = degp.reshape(2, npk, 128)

    pr = 8
    for d in range(1024, 7, -8):
        if npk % d == 0:
            pr = d
            break
    b1r = b1.reshape(1, 32)
    b2r = b2.reshape(1, 32)
    x3 = x.reshape(n // 8, 8, 128)
    dinv_pk, g = _tc_encoder(x3, W1, b1r, W2, b2r, Wc0, degp_pk, pr)

    def bpack(b):
        return jnp.tile(b.reshape(2, 16), (1, 8)).reshape(1, 2, 128)

    def wdiag(W):
        eye8 = jnp.eye(8, dtype=jnp.float32)
        blocks = [[jnp.kron(eye8, W[16 * u:16 * u + 16, 16 * v:16 * v + 16])
                   for v in range(2)] for u in range(2)]
        return jnp.stack([jnp.stack(r) for r in blocks]).reshape(1, 2, 2, 128, 128)

    for (bc, Wn) in ((bc0, Wc1), (bc1, Wc2)):
        acc = _sc_layer(g.reshape(2 * n_out, 16), srcx_f, dstr_f, zeros_hbm,
                        n_out, rt)
        g = _tc_mid(acc.reshape(2, npk, 128), g, dinv_pk, bpack(bc),
                    wdiag(Wn), pr)

    acc = _sc_layer(g.reshape(2 * n_out, 16), srcx_f, dstr_f, zeros_hbm,
                    n_out, rt)
    out_pk = _tc_final(acc.reshape(2, npk, 128), g, dinv_pk, bpack(bc2), pr)
    out_lin = out_pk.reshape(2, n_out, 16)
    return jnp.concatenate([out_lin[0, :n], out_lin[1, :n]], axis=1)


# speedup vs baseline: 32.5493x; 1.0502x over previous
import functools

import jax
import jax.numpy as jnp
from jax import lax
from jax.experimental import pallas as pl
from jax.experimental.pallas import tpu as pltpu
from jax.experimental.pallas import tpu_sc as plsc

NC = 2
NS = 16
LANE = 128
CH = 6

_HIGH = jax.lax.Precision.HIGHEST


def _mesh():
    return plsc.VectorSubcoreMesh(
        core_axis_name="c", subcore_axis_name="s", num_cores=NC, num_subcores=NS
    )


_SC_PARAMS = pltpu.CompilerParams(use_tc_tiling_on_sc=False)


def _sc_degree(dstr, ones_hbm, zeros_hbm, n_out, zb):
    rt = dstr.shape[0]
    rows_w = rt // (NC * NS)
    nch = rows_w // CH
    zr = n_out // NS
    out_r = n_out // NS
    zfull, zrem = zr // zb, zr % zb

    def body(dstr_h, ones_h, zeros_h, out_h, dst_v, ones_v, zbuf, deg_sh):
        c = lax.axis_index("c")
        s = lax.axis_index("s")
        pltpu.sync_copy(zeros_h, zbuf)
        pltpu.sync_copy(ones_h, ones_v)
        for z in range(zfull):
            pltpu.sync_copy(zbuf, deg_sh.at[pl.ds(s * zr + z * zb, zb)])
        if zrem:
            pltpu.sync_copy(zbuf.at[pl.ds(0, zrem)],
                            deg_sh.at[pl.ds(s * zr + zfull * zb, zrem)])
        plsc.subcore_barrier()
        base = (c * NS + s) * rows_w

        def chunk(i, _):
            pltpu.sync_copy(dstr_h.at[pl.ds(base + i * CH, CH)], dst_v)
            for j in range(CH):
                pltpu.sync_copy(ones_v, deg_sh.at[dst_v.at[j]], add=True)
            return ()

        lax.fori_loop(0, nch, chunk, ())
        plsc.subcore_barrier()
        pltpu.sync_copy(
            deg_sh.at[pl.ds(s * out_r, out_r)], out_h.at[c, pl.ds(s * out_r, out_r)]
        )

    call = pl.kernel(
        body,
        out_type=jax.ShapeDtypeStruct((NC, n_out, 16), jnp.float32),
        mesh=_mesh(),
        scratch_types=[
            pltpu.VMEM((CH, LANE), jnp.int32),
            pltpu.VMEM((LANE, 16), jnp.float32),
            pltpu.VMEM((zb, 16), jnp.float32),
            pltpu.VMEM_SHARED((n_out, 16), jnp.float32),
        ],
        compiler_params=_SC_PARAMS,
    )
    return call(dstr, ones_hbm, zeros_hbm)


CHL = 6


def _sc_layer(g2, srcx_f, dstr_f, zeros_hbm, n_out, rt):
    rows_w = rt // NS
    nch = rows_w // CHL
    assert nch % 4 == 0 and nch >= 8
    ew = CHL * LANE
    zr = n_out // NS
    out_r = n_out // NS
    zb = ew
    zfull, zrem = zr // zb, zr % zb

    def body(g2_h, srcx_h, dstr_h, zeros_h, out_h,
             src_v, dst_v, rows_v, acc_sh, *sems):
        gsem = sems[0:2]
        ssem = sems[2:4]
        isrc = sems[4:6]
        idst = sems[6:10]
        c = lax.axis_index("c")
        s = lax.axis_index("s")
        pltpu.sync_copy(zeros_h, rows_v.at[0])
        for z in range(zfull):
            pltpu.sync_copy(rows_v.at[0], acc_sh.at[pl.ds(s * zr + z * zb, zb)])
        if zrem:
            pltpu.sync_copy(rows_v.at[0, pl.ds(0, zrem)],
                            acc_sh.at[pl.ds(s * zr + zfull * zb, zrem)])
        plsc.subcore_barrier()
        ebase = s * rows_w * LANE

        def load_src(k, slot, sync=False):
            sl = srcx_h.at[c, pl.ds(ebase + k * ew, ew)]
            if sync:
                pltpu.sync_copy(sl, src_v.at[slot])
            else:
                pltpu.async_copy(sl, src_v.at[slot], isrc[slot])

        def load_dst(k, slot, sync=False):
            sl = dstr_h.at[pl.ds(ebase + k * ew, ew)]
            if sync:
                pltpu.sync_copy(sl, dst_v.at[slot])
            else:
                pltpu.async_copy(sl, dst_v.at[slot], idst[slot])

        def wait_isrc(slot):
            pltpu.make_async_copy(
                srcx_h.at[c, pl.ds(0, ew)], src_v.at[slot], isrc[slot]).wait()

        def wait_idst(slot):
            pltpu.make_async_copy(
                dstr_h.at[pl.ds(0, ew)], dst_v.at[slot], idst[slot]).wait()

        def fire_g(b):
            for j in range(CHL):
                pltpu.async_copy(
                    g2_h.at[src_v.at[b, pl.ds(j * LANE, LANE)]],
                    rows_v.at[b, pl.ds(j * LANE, LANE)], gsem[b])

        def wait_g(b):
            for j in range(CHL):
                pltpu.make_async_copy(
                    g2_h.at[src_v.at[b, pl.ds(j * LANE, LANE)]],
                    rows_v.at[b, pl.ds(j * LANE, LANE)], gsem[b]).wait()

        def fire_s(b, d):
            for j in range(CHL):
                pltpu.async_copy(
                    rows_v.at[b, pl.ds(j * LANE, LANE)],
                    acc_sh.at[dst_v.at[d, pl.ds(j * LANE, LANE)]],
                    ssem[b], add=True)

        def wait_s(b, d):
            for j in range(CHL):
                pltpu.make_async_copy(
                    rows_v.at[b, pl.ds(j * LANE, LANE)],
                    acc_sh.at[dst_v.at[d, pl.ds(j * LANE, LANE)]],
                    ssem[b]).wait()

        def visit_steady(k, b, b1, d, d2, first=False, second=False):
            if not first:
                wait_s(b1, (d + 3) % 4)
            if not first:
                wait_isrc(b1)
            fire_g(b1)
            wait_g(b)
            load_src(k + 2, b)
            if not (first or second):
                wait_idst(d)
            load_dst(k + 2, d2)
            fire_s(b, d)

        load_src(0, 0, sync=True)
        load_src(1, 1, sync=True)
        load_dst(0, 0, sync=True)
        load_dst(1, 1, sync=True)
        fire_g(0)
        visit_steady(0, 0, 1, 0, 2, first=True)
        visit_steady(1, 1, 0, 1, 3, second=True)
        visit_steady(2, 0, 1, 2, 0)
        visit_steady(3, 1, 0, 3, 1)

        def round_body(r, _):
            k = r * 4
            visit_steady(k + 0, 0, 1, 0, 2)
            visit_steady(k + 1, 1, 0, 1, 3)
            visit_steady(k + 2, 0, 1, 2, 0)
            visit_steady(k + 3, 1, 0, 3, 1)
            return ()

        lax.fori_loop(1, nch // 4, round_body, ())
        wait_g(0)
        wait_s(1, 3)
        wait_isrc(1)
        wait_idst(0)
        wait_idst(1)
        plsc.subcore_barrier()
        pltpu.sync_copy(
            acc_sh.at[pl.ds(s * out_r, out_r)], out_h.at[c, pl.ds(s * out_r, out_r)]
        )

    call = pl.kernel(
        body,
        out_type=jax.ShapeDtypeStruct((NC, n_out, 16), jnp.float32),
        mesh=_mesh(),
        scratch_types=[
            pltpu.VMEM((2, CHL * LANE), jnp.int32),
            pltpu.VMEM((4, CHL * LANE), jnp.int32),
            pltpu.VMEM((2, CHL * LANE, 16), jnp.float32),
            pltpu.VMEM_SHARED((n_out, 16), jnp.float32),
        ] + [pltpu.SemaphoreType.DMA] * 10,
        compiler_params=_SC_PARAMS,
    )
    return call(g2, srcx_f, dstr_f, zeros_hbm)


def _tc_prep_idx(src_r, n_nodes):
    rt = src_r.shape[0]
    blk = 8
    for d in range(2048, 7, -8):
        if rt % d == 0:
            blk = d
            break

    def body(s_ref, o_ref):
        v = s_ref[...]
        o_ref[0] = v
        o_ref[1] = v + n_nodes

    return pl.pallas_call(
        body,
        grid=(rt // blk,),
        in_specs=[pl.BlockSpec((blk, LANE), lambda i: (i, 0))],
        out_specs=pl.BlockSpec((2, blk, LANE), lambda i: (0, i, 0)),
        out_shape=jax.ShapeDtypeStruct((2, rt, LANE), jnp.int32),
    )(src_r)


def _tc_encoder(xs, W1, b1, W2, b2, Wc0, degp_pk, pr):
    npk = degp_pk.shape[1]

    def body(*refs):
        x_refs = refs[:8]
        w1_ref, b1_ref, w2d_ref, b2t_ref, k_ref, d0_ref, d1_ref = refs[8:15]
        dinv_ref, g_ref = refs[15:]
        dinv = lax.rsqrt(d0_ref[0] + d1_ref[0] + 1.0)
        t = jnp.concatenate(
            [jnp.maximum(jnp.dot(x_refs[a][...], w1_ref[...], precision=_HIGH)
                         + b1_ref[...], 0.0)
             for a in range(8)], axis=1)
        h = jnp.dot(t, w2d_ref[...], precision=_HIGH) + b2t_ref[...]
        dinv_ref[...] = dinv
        g_ref[0] = jnp.dot(h, k_ref[0], precision=_HIGH) * dinv
        g_ref[1] = jnp.dot(h, k_ref[1], precision=_HIGH) * dinv

    return pl.pallas_call(
        body,
        grid=(npk // pr,),
        in_specs=[pl.BlockSpec((pr, 128), lambda i: (i, 0)) for _ in range(8)]
        + [
            pl.BlockSpec((128, 32), lambda i: (0, 0)),
            pl.BlockSpec((1, 32), lambda i: (0, 0)),
            pl.BlockSpec((256, 256), lambda i: (0, 0)),
            pl.BlockSpec((1, 256), lambda i: (0, 0)),
            pl.BlockSpec((2, 256, 128), lambda i: (0, 0, 0)),
            pl.BlockSpec((1, pr, 128), lambda i: (0, i, 0)),
            pl.BlockSpec((1, pr, 128), lambda i: (1, i, 0)),
        ],
        out_specs=[
            pl.BlockSpec((pr, 128), lambda i: (i, 0)),
            pl.BlockSpec((2, pr, 128), lambda i: (0, i, 0)),
        ],
        out_shape=[
            jax.ShapeDtypeStruct((npk, 128), jnp.float32),
            jax.ShapeDtypeStruct((2, npk, 128), jnp.float32),
        ],
    )(*xs, W1, b1, W2, b2, Wc0, degp_pk, degp_pk)


def _tc_mid(acc_pk, g_pk, dinv_pk, b_pk, Wd, pr):
    npk = dinv_pk.shape[0]
    n_steps = npk // pr

    def body(a0, a1, g0, g1, d_ref, b_ref, wd_ref, o_ref):
        dinv = d_ref[...]
        h0 = jnp.maximum(dinv * (a0[0] + g0[0]) + b_ref[0, 0:1, :], 0.0)
        h1 = jnp.maximum(dinv * (a1[0] + g1[0]) + b_ref[0, 1:2, :], 0.0)
        for v in range(2):
            gn = (jnp.dot(h0, wd_ref[0, 0, v], precision=_HIGH)
                  + jnp.dot(h1, wd_ref[0, 1, v], precision=_HIGH)) * dinv
            o_ref[v] = gn

    return pl.pallas_call(
        body,
        grid=(n_steps,),
        in_specs=[
            pl.BlockSpec((1, pr, 128), lambda i: (0, i, 0)),
            pl.BlockSpec((1, pr, 128), lambda i: (1, i, 0)),
            pl.BlockSpec((1, pr, 128), lambda i: (0, i, 0)),
            pl.BlockSpec((1, pr, 128), lambda i: (1, i, 0)),
            pl.BlockSpec((pr, 128), lambda i: (i, 0)),
            pl.BlockSpec((1, 2, 128), lambda i: (0, 0, 0)),
            pl.BlockSpec((1, 2, 2, 128, 128), lambda i: (0, 0, 0, 0, 0)),
        ],
        out_specs=pl.BlockSpec((2, pr, 128), lambda i: (0, i, 0)),
        out_shape=jax.ShapeDtypeStruct((2, npk, 128), jnp.float32),
    )(acc_pk, acc_pk, g_pk, g_pk, dinv_pk, b_pk, Wd)


def _tc_final(acc_pk, g_pk, dinv_pk, b_pk, pr):
    npk = acc_pk.shape[1]

    def body(a0, a1, g0, g1, d_ref, b_ref, o_ref):
        dinv = d_ref[...]
        o_ref[0] = jnp.maximum(dinv * (a0[0] + g0[0]) + b_ref[0, 0:1, :], 0.0)
        o_ref[1] = jnp.maximum(dinv * (a1[0] + g1[0]) + b_ref[0, 1:2, :], 0.0)

    return pl.pallas_call(
        body,
        grid=(npk // pr,),
        in_specs=[
            pl.BlockSpec((1, pr, 128), lambda i: (0, i, 0)),
            pl.BlockSpec((1, pr, 128), lambda i: (1, i, 0)),
            pl.BlockSpec((1, pr, 128), lambda i: (0, i, 0)),
            pl.BlockSpec((1, pr, 128), lambda i: (1, i, 0)),
            pl.BlockSpec((pr, 128), lambda i: (i, 0)),
            pl.BlockSpec((1, 2, 128), lambda i: (0, 0, 0)),
        ],
        out_specs=pl.BlockSpec((2, pr, 128), lambda i: (0, i, 0)),
        out_shape=jax.ShapeDtypeStruct((2, npk, 128), jnp.float32),
    )(acc_pk, acc_pk, g_pk, g_pk, dinv_pk, b_pk)


def kernel(x, edge_index, W1, b1, W2, b2, Wc0, bc0, Wc1, bc1, Wc2, bc2):
    n, f_in = x.shape
    e = edge_index.shape[1]
    assert f_in == 128 and W1.shape[1] == 32
    assert n % NS == 0

    rows = -(-e // LANE)
    rt = -(-rows // (NS * CHL * 4)) * (NS * CHL * 4)
    assert (rt // (NC * NS)) % CH == 0
    rtp = rt + 16
    pad = rtp * LANE - e

    n_out = -(-(n + 1) // (NS * 8)) * (NS * 8)
    npk = n_out // 8

    src = jnp.concatenate([edge_index[0], jnp.zeros((pad,), jnp.int32)])
    dst = jnp.concatenate([edge_index[1], jnp.full((pad,), n, jnp.int32)])
    src_r = src.reshape(rtp, LANE)
    dstr = dst[: rt * LANE].reshape(rt, LANE)
    srcx = _tc_prep_idx(src_r, n_out)
    srcx_f = srcx.reshape(2, rtp * LANE)
    dstr_f = dst

    zb = CHL * LANE
    zeros_hbm = jnp.zeros((zb, 16), jnp.float32)
    ones_hbm = jnp.ones((LANE, 16), jnp.float32)

    degp = _sc_degree(dstr, ones_hbm, zeros_hbm, n_out, zb)
    degp_pk = degp.reshape(2, npk, 128)

    pr = 8
    for d in range(1024, 7, -8):
        if npk % d == 0:
            pr = d
            break
    b1r = b1.reshape(1, 32)
    eye8 = jnp.eye(8, dtype=jnp.float32)
    W2d = jnp.kron(eye8, W2)
    b2t = jnp.tile(b2, 8).reshape(1, 256)
    Kv = jnp.stack([jnp.kron(eye8, Wc0[:, :16]), jnp.kron(eye8, Wc0[:, 16:])])
    x3 = x.reshape(n // 8, 8, 128)
    xs = [x3[:, a, :] for a in range(8)]
    dinv_pk, g = _tc_encoder(xs, W1, b1r, W2d, b2t, Kv, degp_pk, pr)

    def bpack(b):
        return jnp.tile(b.reshape(2, 16), (1, 8)).reshape(1, 2, 128)

    def wdiag(W):
        eye8 = jnp.eye(8, dtype=jnp.float32)
        blocks = [[jnp.kron(eye8, W[16 * u:16 * u + 16, 16 * v:16 * v + 16])
                   for v in range(2)] for u in range(2)]
        return jnp.stack([jnp.stack(r) for r in blocks]).reshape(1, 2, 2, 128, 128)

    for (bc, Wn) in ((bc0, Wc1), (bc1, Wc2)):
        acc = _sc_layer(g.reshape(2 * n_out, 16), srcx_f, dstr_f, zeros_hbm,
                        n_out, rt)
        g = _tc_mid(acc.reshape(2, npk, 128), g, dinv_pk, bpack(bc),
                    wdiag(Wn), pr)

    acc = _sc_layer(g.reshape(2 * n_out, 16), srcx_f, dstr_f, zeros_hbm,
                    n_out, rt)
    out_pk = _tc_final(acc.reshape(2, npk, 128), g, dinv_pk, bpack(bc2), pr)
    out_lin = out_pk.reshape(2, n_out, 16)[:, :n]
    return out_lin.transpose(1, 0, 2).reshape(n, 32)


# speedup vs baseline: 32.5703x; 1.0006x over previous
import functools

import jax
import jax.numpy as jnp
from jax import lax
from jax.experimental import pallas as pl
from jax.experimental.pallas import tpu as pltpu
from jax.experimental.pallas import tpu_sc as plsc

NC = 2
NS = 16
LANE = 128
CH = 6
CHL = 6

_HIGH = jax.lax.Precision.HIGHEST


def _mesh():
    return plsc.VectorSubcoreMesh(
        core_axis_name="c", subcore_axis_name="s", num_cores=NC, num_subcores=NS
    )


_SC_PARAMS = pltpu.CompilerParams(use_tc_tiling_on_sc=False)


def _sc_degree(dstr, ones_hbm, zeros_hbm, n_out, zb, rt):
    rows_w = rt // (NC * NS)
    nch = rows_w // CH
    assert nch % 4 == 2 or nch % 4 == 0
    assert (nch - 2) % 4 == 0 and nch >= 6
    zr = n_out // NS
    out_r = n_out // NS
    zfull, zrem = zr // zb, zr % zb
    ew = CH * LANE

    def body(dstr_h, ones_h, zeros_h, out_h, dst_v, ones_v, zbuf, deg_sh, *sems):
        ssem = sems[0:2]
        idst = sems[2:6]
        c = lax.axis_index("c")
        s = lax.axis_index("s")
        pltpu.sync_copy(zeros_h, zbuf)
        pltpu.sync_copy(ones_h, ones_v)
        for z in range(zfull):
            pltpu.sync_copy(zbuf, deg_sh.at[pl.ds(s * zr + z * zb, zb)])
        if zrem:
            pltpu.sync_copy(zbuf.at[pl.ds(0, zrem)],
                            deg_sh.at[pl.ds(s * zr + zfull * zb, zrem)])
        plsc.subcore_barrier()
        base = (c * NS + s) * rows_w

        def load_dst(k, slot, sync=False):
            sl = dstr_h.at[pl.ds(base + k * CH, CH)]
            if sync:
                pltpu.sync_copy(sl, dst_v.at[slot])
            else:
                pltpu.async_copy(sl, dst_v.at[slot], idst[slot])

        def wait_idst(slot):
            pltpu.make_async_copy(
                dstr_h.at[pl.ds(0, CH)], dst_v.at[slot], idst[slot]).wait()

        def fire_s(b, d):
            for j in range(CH):
                pltpu.async_copy(
                    ones_v,
                    deg_sh.at[dst_v.at[d, j]],
                    ssem[b], add=True)

        def wait_s(b, d):
            for j in range(CH):
                pltpu.make_async_copy(
                    ones_v, deg_sh.at[dst_v.at[d, j]], ssem[b]).wait()

        def visit(k, b, d, d2, first=False):
            if not first:
                wait_idst(d)
                wait_s(b, (d + 2) % 4)
            load_dst(k + 2, d2)
            fire_s(b, d)

        load_dst(0, 0, sync=True)
        load_dst(1, 1, sync=True)
        visit(0, 0, 0, 2, first=True)
        visit(1, 1, 1, 3, first=True)

        def round_body(r, _):
            k = r * 4 + 2
            visit(k + 0, 0, 2, 0)
            visit(k + 1, 1, 3, 1)
            visit(k + 2, 0, 0, 2)
            visit(k + 3, 1, 1, 3)
            return ()

        lax.fori_loop(0, (nch - 2) // 4, round_body, ())
        wait_s(0, (nch - 2) % 4)
        wait_s(1, (nch - 1) % 4)
        wait_idst(nch % 4)
        wait_idst((nch + 1) % 4)
        plsc.subcore_barrier()
        pltpu.sync_copy(
            deg_sh.at[pl.ds(s * out_r, out_r)], out_h.at[c, pl.ds(s * out_r, out_r)]
        )

    call = pl.kernel(
        body,
        out_type=jax.ShapeDtypeStruct((NC, n_out, 16), jnp.float32),
        mesh=_mesh(),
        scratch_types=[
            pltpu.VMEM((4, CH, LANE), jnp.int32),
            pltpu.VMEM((LANE, 16), jnp.float32),
            pltpu.VMEM((zb, 16), jnp.float32),
            pltpu.VMEM_SHARED((n_out, 16), jnp.float32),
        ] + [pltpu.SemaphoreType.DMA] * 6,
        compiler_params=_SC_PARAMS,
    )
    return call(dstr, ones_hbm, zeros_hbm)


def _sc_layer(g2, srcx_f, dstr_f, zeros_hbm, n_out, rt):
    rows_w = rt // NS
    nch = rows_w // CHL
    assert nch % 4 == 0 and nch >= 8
    ew = CHL * LANE
    zr = n_out // NS
    out_r = n_out // NS
    zb = ew
    zfull, zrem = zr // zb, zr % zb

    def body(g2_h, srcx_h, dstr_h, zeros_h, out_h,
             src_v, dst_v, rows_v, acc_sh, *sems):
        gsem = sems[0:2]
        ssem = sems[2:4]
        isrc = sems[4:6]
        idst = sems[6:10]
        c = lax.axis_index("c")
        s = lax.axis_index("s")
        pltpu.sync_copy(zeros_h, rows_v.at[0])
        for z in range(zfull):
            pltpu.sync_copy(rows_v.at[0], acc_sh.at[pl.ds(s * zr + z * zb, zb)])
        if zrem:
            pltpu.sync_copy(rows_v.at[0, pl.ds(0, zrem)],
                            acc_sh.at[pl.ds(s * zr + zfull * zb, zrem)])
        plsc.subcore_barrier()
        ebase = s * rows_w * LANE

        def load_src(k, slot, sync=False):
            sl = srcx_h.at[c, pl.ds(ebase + k * ew, ew)]
            if sync:
                pltpu.sync_copy(sl, src_v.at[slot])
            else:
                pltpu.async_copy(sl, src_v.at[slot], isrc[slot])

        def load_dst(k, slot, sync=False):
            sl = dstr_h.at[pl.ds(ebase + k * ew, ew)]
            if sync:
                pltpu.sync_copy(sl, dst_v.at[slot])
            else:
                pltpu.async_copy(sl, dst_v.at[slot], idst[slot])

        def wait_isrc(slot):
            pltpu.make_async_copy(
                srcx_h.at[c, pl.ds(0, ew)], src_v.at[slot], isrc[slot]).wait()

        def wait_idst(slot):
            pltpu.make_async_copy(
                dstr_h.at[pl.ds(0, ew)], dst_v.at[slot], idst[slot]).wait()

        def fire_g(b):
            for j in range(CHL):
                pltpu.async_copy(
                    g2_h.at[src_v.at[b, pl.ds(j * LANE, LANE)]],
                    rows_v.at[b, pl.ds(j * LANE, LANE)], gsem[b])

        def wait_g(b):
            for j in range(CHL):
                pltpu.make_async_copy(
                    g2_h.at[src_v.at[b, pl.ds(j * LANE, LANE)]],
                    rows_v.at[b, pl.ds(j * LANE, LANE)], gsem[b]).wait()

        def fire_s(b, d):
            for j in range(CHL):
                pltpu.async_copy(
                    rows_v.at[b, pl.ds(j * LANE, LANE)],
                    acc_sh.at[dst_v.at[d, pl.ds(j * LANE, LANE)]],
                    ssem[b], add=True)

        def wait_s(b, d):
            for j in range(CHL):
                pltpu.make_async_copy(
                    rows_v.at[b, pl.ds(j * LANE, LANE)],
                    acc_sh.at[dst_v.at[d, pl.ds(j * LANE, LANE)]],
                    ssem[b]).wait()

        def visit_steady(k, b, b1, d, d2, first=False, second=False):
            if not first:
                wait_s(b1, (d + 3) % 4)
            if not first:
                wait_isrc(b1)
            fire_g(b1)
            wait_g(b)
            load_src(k + 2, b)
            if not (first or second):
                wait_idst(d)
            load_dst(k + 2, d2)
            fire_s(b, d)

        load_src(0, 0, sync=True)
        load_src(1, 1, sync=True)
        load_dst(0, 0, sync=True)
        load_dst(1, 1, sync=True)
        fire_g(0)
        visit_steady(0, 0, 1, 0, 2, first=True)
        visit_steady(1, 1, 0, 1, 3, second=True)
        visit_steady(2, 0, 1, 2, 0)
        visit_steady(3, 1, 0, 3, 1)

        def round_body(r, _):
            k = r * 4
            visit_steady(k + 0, 0, 1, 0, 2)
            visit_steady(k + 1, 1, 0, 1, 3)
            visit_steady(k + 2, 0, 1, 2, 0)
            visit_steady(k + 3, 1, 0, 3, 1)
            return ()

        lax.fori_loop(1, nch // 4, round_body, ())
        wait_g(0)
        wait_s(1, 3)
        wait_isrc(1)
        wait_idst(0)
        wait_idst(1)
        plsc.subcore_barrier()
        pltpu.sync_copy(
            acc_sh.at[pl.ds(s * out_r, out_r)], out_h.at[c, pl.ds(s * out_r, out_r)]
        )

    call = pl.kernel(
        body,
        out_type=jax.ShapeDtypeStruct((NC, n_out, 16), jnp.float32),
        mesh=_mesh(),
        scratch_types=[
            pltpu.VMEM((2, CHL * LANE), jnp.int32),
            pltpu.VMEM((4, CHL * LANE), jnp.int32),
            pltpu.VMEM((2, CHL * LANE, 16), jnp.float32),
            pltpu.VMEM_SHARED((n_out, 16), jnp.float32),
        ] + [pltpu.SemaphoreType.DMA] * 10,
        compiler_params=_SC_PARAMS,
    )
    return call(g2, srcx_f, dstr_f, zeros_hbm)


def _tc_prep_idx(src_r, n_nodes):
    rt = src_r.shape[0]
    blk = 8
    for d in range(2048, 7, -8):
        if rt % d == 0:
            blk = d
            break

    def body(s_ref, o_ref):
        v = s_ref[...]
        o_ref[0] = v
        o_ref[1] = v + n_nodes

    return pl.pallas_call(
        body,
        grid=(rt // blk,),
        in_specs=[pl.BlockSpec((blk, LANE), lambda i: (i, 0))],
        out_specs=pl.BlockSpec((2, blk, LANE), lambda i: (0, i, 0)),
        out_shape=jax.ShapeDtypeStruct((2, rt, LANE), jnp.int32),
    )(src_r)


def _tc_encoder(xs, W1, b1, W2, b2, Wc0, degp_pk, pr):
    npk = degp_pk.shape[1]

    def body(*refs):
        x_refs = refs[:8]
        w1_ref, b1_ref, w2d_ref, b2t_ref, k_ref, d0_ref, d1_ref = refs[8:15]
        dinv_ref, g_ref = refs[15:]
        dinv = lax.rsqrt(d0_ref[0] + d1_ref[0] + 1.0)
        t = jnp.concatenate(
            [jnp.maximum(jnp.dot(x_refs[a][...], w1_ref[...], precision=_HIGH)
                         + b1_ref[...], 0.0)
             for a in range(8)], axis=1)
        h = jnp.dot(t, w2d_ref[...], precision=_HIGH) + b2t_ref[...]
        dinv_ref[...] = dinv
        g_ref[0] = jnp.dot(h, k_ref[0], precision=_HIGH) * dinv
        g_ref[1] = jnp.dot(h, k_ref[1], precision=_HIGH) * dinv

    return pl.pallas_call(
        body,
        grid=(npk // pr,),
        in_specs=[pl.BlockSpec((pr, 128), lambda i: (i, 0)) for _ in range(8)]
        + [
            pl.BlockSpec((128, 32), lambda i: (0, 0)),
            pl.BlockSpec((1, 32), lambda i: (0, 0)),
            pl.BlockSpec((256, 256), lambda i: (0, 0)),
            pl.BlockSpec((1, 256), lambda i: (0, 0)),
            pl.BlockSpec((2, 256, 128), lambda i: (0, 0, 0)),
            pl.BlockSpec((1, pr, 128), lambda i: (0, i, 0)),
            pl.BlockSpec((1, pr, 128), lambda i: (1, i, 0)),
        ],
        out_specs=[
            pl.BlockSpec((pr, 128), lambda i: (i, 0)),
            pl.BlockSpec((2, pr, 128), lambda i: (0, i, 0)),
        ],
        out_shape=[
            jax.ShapeDtypeStruct((npk, 128), jnp.float32),
            jax.ShapeDtypeStruct((2, npk, 128), jnp.float32),
        ],
    )(*xs, W1, b1, W2, b2, Wc0, degp_pk, degp_pk)


def _tc_mid(acc_pk, g_pk, dinv_pk, b_pk, Wd, pr):
    npk = dinv_pk.shape[0]
    n_steps = npk // pr

    def body(a0, a1, g0, g1, d_ref, b_ref, wd_ref, o_ref):
        dinv = d_ref[...]
        h0 = jnp.maximum(dinv * (a0[0] + g0[0]) + b_ref[0, 0:1, :], 0.0)
        h1 = jnp.maximum(dinv * (a1[0] + g1[0]) + b_ref[0, 1:2, :], 0.0)
        for v in range(2):
            gn = (jnp.dot(h0, wd_ref[0, 0, v], precision=_HIGH)
                  + jnp.dot(h1, wd_ref[0, 1, v], precision=_HIGH)) * dinv
            o_ref[v] = gn

    return pl.pallas_call(
        body,
        grid=(n_steps,),
        in_specs=[
            pl.BlockSpec((1, pr, 128), lambda i: (0, i, 0)),
            pl.BlockSpec((1, pr, 128), lambda i: (1, i, 0)),
            pl.BlockSpec((1, pr, 128), lambda i: (0, i, 0)),
            pl.BlockSpec((1, pr, 128), lambda i: (1, i, 0)),
            pl.BlockSpec((pr, 128), lambda i: (i, 0)),
            pl.BlockSpec((1, 2, 128), lambda i: (0, 0, 0)),
            pl.BlockSpec((1, 2, 2, 128, 128), lambda i: (0, 0, 0, 0, 0)),
        ],
        out_specs=pl.BlockSpec((2, pr, 128), lambda i: (0, i, 0)),
        out_shape=jax.ShapeDtypeStruct((2, npk, 128), jnp.float32),
    )(acc_pk, acc_pk, g_pk, g_pk, dinv_pk, b_pk, Wd)


def _tc_final(acc_pk, g_pk, dinv_pk, b_pk, pr):
    npk = acc_pk.shape[1]

    def body(a0, a1, g0, g1, d_ref, b_ref, o_ref):
        dinv = d_ref[...]
        o_ref[0] = jnp.maximum(dinv * (a0[0] + g0[0]) + b_ref[0, 0:1, :], 0.0)
        o_ref[1] = jnp.maximum(dinv * (a1[0] + g1[0]) + b_ref[0, 1:2, :], 0.0)

    return pl.pallas_call(
        body,
        grid=(npk // pr,),
        in_specs=[
            pl.BlockSpec((1, pr, 128), lambda i: (0, i, 0)),
            pl.BlockSpec((1, pr, 128), lambda i: (1, i, 0)),
            pl.BlockSpec((1, pr, 128), lambda i: (0, i, 0)),
            pl.BlockSpec((1, pr, 128), lambda i: (1, i, 0)),
            pl.BlockSpec((pr, 128), lambda i: (i, 0)),
            pl.BlockSpec((1, 2, 128), lambda i: (0, 0, 0)),
        ],
        out_specs=pl.BlockSpec((2, pr, 128), lambda i: (0, i, 0)),
        out_shape=jax.ShapeDtypeStruct((2, npk, 128), jnp.float32),
    )(acc_pk, acc_pk, g_pk, g_pk, dinv_pk, b_pk)


def kernel(x, edge_index, W1, b1, W2, b2, Wc0, bc0, Wc1, bc1, Wc2, bc2):
    n, f_in = x.shape
    e = edge_index.shape[1]
    assert f_in == 128 and W1.shape[1] == 32
    assert n % NS == 0

    rows = -(-e // LANE)
    rt = -(-rows // (NS * CHL * 4)) * (NS * CHL * 4)
    assert (rt // (NC * NS)) % CH == 0
    rtp = rt + 16
    pad = rtp * LANE - e

    n_out = -(-(n + 1) // (NS * 8)) * (NS * 8)
    npk = n_out // 8

    src = jnp.concatenate([edge_index[0], jnp.zeros((pad,), jnp.int32)])
    dst = jnp.concatenate([edge_index[1], jnp.full((pad,), n, jnp.int32)])
    src_r = src.reshape(rtp, LANE)
    dstr = dst.reshape(rtp, LANE)
    srcx = _tc_prep_idx(src_r, n_out)
    srcx_f = srcx.reshape(2, rtp * LANE)
    dstr_f = dst

    zb = CHL * LANE
    zeros_hbm = jnp.zeros((zb, 16), jnp.float32)
    ones_hbm = jnp.ones((LANE, 16), jnp.float32)

    degp = _sc_degree(dstr, ones_hbm, zeros_hbm, n_out, zb, rt)
    degp_pk = degp.reshape(2, npk, 128)

    pr = 8
    for d in range(1024, 7, -8):
        if npk % d == 0:
            pr = d
            break
    b1r = b1.reshape(1, 32)
    eye8 = jnp.eye(8, dtype=jnp.float32)
    W2d = jnp.kron(eye8, W2)
    b2t = jnp.tile(b2, 8).reshape(1, 256)
    Kv = jnp.stack([jnp.kron(eye8, Wc0[:, :16]), jnp.kron(eye8, Wc0[:, 16:])])
    x3 = x.reshape(n // 8, 8, 128)
    xs = [x3[:, a, :] for a in range(8)]
    dinv_pk, g = _tc_encoder(xs, W1, b1r, W2d, b2t, Kv, degp_pk, pr)

    def bpack(b):
        return jnp.tile(b.reshape(2, 16), (1, 8)).reshape(1, 2, 128)

    def wdiag(W):
        eye8 = jnp.eye(8, dtype=jnp.float32)
        blocks = [[jnp.kron(eye8, W[16 * u:16 * u + 16, 16 * v:16 * v + 16])
                   for v in range(2)] for u in range(2)]
        return jnp.stack([jnp.stack(r) for r in blocks]).reshape(1, 2, 2, 128, 128)

    for (bc, Wn) in ((bc0, Wc1), (bc1, Wc2)):
        acc = _sc_layer(g.reshape(2 * n_out, 16), srcx_f, dstr_f, zeros_hbm,
                        n_out, rt)
        g = _tc_mid(acc.reshape(2, npk, 128), g, dinv_pk, bpack(bc),
                    wdiag(Wn), pr)

    acc = _sc_layer(g.reshape(2 * n_out, 16), srcx_f, dstr_f, zeros_hbm,
                    n_out, rt)
    out_pk = _tc_final(acc.reshape(2, npk, 128), g, dinv_pk, bpack(bc2), pr)
    out_lin = out_pk.reshape(2, n_out, 16)[:, :n]
    return out_lin.transpose(1, 0, 2).reshape(n, 32)
